# Initial kernel scaffold; baseline (speedup 1.0000x reference)
import jax
import jax.numpy as jnp
from jax.experimental import pallas as pl


def kernel(events, times, z_rows, z_cols, gamma_rows, gamma_cols, b):
    raise NotImplementedError("write your pallas kernel here")


# speedup vs baseline: 998.1849x; 998.1849x over previous
import functools

import jax
import jax.numpy as jnp
from jax.experimental import pallas as pl

ROW_SIZE = 10000
COL_SIZE = 200
DIM = 16
BLK = 1000


def _nll_kernel(ev_ref, z_ref, grow_ref, ycross_ref, colq_ref, out_ref):
    z = z_ref[...]
    a = z[:, 0:DIM]
    bb = z[:, DIM:2 * DIM]
    c = z[:, 2 * DIM:3 * DIM]
    na = jnp.sum(a * a, axis=1, keepdims=True)
    nb = jnp.sum(bb * bb, axis=1, keepdims=True)
    nc = jnp.sum(c * c, axis=1, keepdims=True)
    ab = jnp.sum(a * bb, axis=1, keepdims=True)
    ac = jnp.sum(a * c, axis=1, keepdims=True)
    bc = jnp.sum(bb * c, axis=1, keepdims=True)

    cq = colq_ref[...]
    cross = jax.lax.dot_general(
        z, ycross_ref[...], (((1,), (0,)), ((), ())),
        preferred_element_type=jnp.float32,
        precision=jax.lax.Precision.HIGHEST)
    q = (na + nb * cq[0:1, :] + nc * cq[1:2, :]
         + ab * cq[2:3, :] + ac * cq[3:4, :] + bc * cq[4:5, :]
         + cq[5:6, :])
    dist = jnp.sqrt(jnp.maximum(cross + q, 0.0))
    f = grow_ref[...] + cq[6:7, :] - dist

    e = ev_ref[...]
    b0 = cq[7:8, :]
    b1 = cq[8:9, :]
    b2 = cq[9:10, :]
    inv_sqrt2 = 0.7071067811865476
    th_hi = jnp.where(e == 1, b0, jnp.where(e == 2, b1, b2))
    th_lo = jnp.where(e == 2, b0, b1)
    phi_hi = 0.5 * (1.0 + jax.lax.erf((th_hi - f) * inv_sqrt2))
    phi_lo = 0.5 * (1.0 + jax.lax.erf((th_lo - f) * inv_sqrt2))
    p = phi_hi - jnp.where(e == 1, 0.0, phi_lo)
    ll = jnp.where(e == 0, 0.0, jnp.log(p))
    partial = -jnp.sum(ll, axis=(0, 1), keepdims=True)

    @pl.when(pl.program_id(0) == 0)
    def _init():
        out_ref[...] = partial

    @pl.when(pl.program_id(0) != 0)
    def _acc():
        out_ref[...] += partial


@functools.partial(jax.jit, static_argnames=())
def kernel(events, times, z_rows, z_cols, gamma_rows, gamma_cols, b):
    t = times.astype(jnp.float32)
    s = 0.5 * t * t
    w = z_cols - 1e-6
    wt = w.T
    ycross = jnp.concatenate(
        [-2.0 * wt, (-2.0 * t)[None, :] * wt, (-2.0 * s)[None, :] * wt],
        axis=0)
    nw = jnp.sum(w * w, axis=1)
    colq = jnp.stack([
        t * t, s * s, 2.0 * t, 2.0 * s, 2.0 * t * s, nw,
        gamma_cols,
        jnp.full_like(t, b[0]), jnp.full_like(t, b[1]), jnp.full_like(t, b[2]),
        jnp.zeros_like(t), jnp.zeros_like(t), jnp.zeros_like(t),
        jnp.zeros_like(t), jnp.zeros_like(t), jnp.zeros_like(t),
    ], axis=0)
    z48 = jnp.transpose(z_rows, (1, 0, 2)).reshape(ROW_SIZE, 3 * DIM)
    grow = gamma_rows.reshape(ROW_SIZE, 1)

    out = pl.pallas_call(
        _nll_kernel,
        grid=(ROW_SIZE // BLK,),
        in_specs=[
            pl.BlockSpec((BLK, COL_SIZE), lambda i: (i, 0)),
            pl.BlockSpec((BLK, 3 * DIM), lambda i: (i, 0)),
            pl.BlockSpec((BLK, 1), lambda i: (i, 0)),
            pl.BlockSpec((3 * DIM, COL_SIZE), lambda i: (0, 0)),
            pl.BlockSpec((16, COL_SIZE), lambda i: (0, 0)),
        ],
        out_specs=pl.BlockSpec((1, 1), lambda i: (0, 0)),
        out_shape=jax.ShapeDtypeStruct((1, 1), jnp.float32),
    )(events, z48, grow, ycross, colq)
    return out[0, 0]


# speedup vs baseline: 1194.7931x; 1.1970x over previous
import functools

import jax
import jax.numpy as jnp
from jax.experimental import pallas as pl

ROW_SIZE = 10000
COL_SIZE = 200
DIM = 16
BLK = 2000

_INV_SQRT2 = 0.7071067811865476


def _nll_kernel(ev_ref, t_ref, z_ref, zc_ref, grow_ref, gcol_ref, b_ref,
                out_ref):
    z = z_ref[...]
    a = z[:, 0:DIM]
    bb = z[:, DIM:2 * DIM]
    c = z[:, 2 * DIM:3 * DIM]
    na = jnp.sum(a * a, axis=1, keepdims=True)
    nb = jnp.sum(bb * bb, axis=1, keepdims=True)
    nc = jnp.sum(c * c, axis=1, keepdims=True)
    ab = jnp.sum(a * bb, axis=1, keepdims=True)
    ac = jnp.sum(a * c, axis=1, keepdims=True)
    bc = jnp.sum(bb * c, axis=1, keepdims=True)

    t = t_ref[...]
    s = 0.5 * t * t
    wp = zc_ref[...] - 1e-6
    y = jnp.concatenate([-2.0 * wp, (-2.0 * t) * wp, (-2.0 * s) * wp],
                        axis=0)
    dims = (((1,), (0,)), ((), ()))
    cross = jax.lax.dot_general(
        z, y, dims, preferred_element_type=jnp.float32,
        precision=jax.lax.Precision.HIGHEST)
    nw = jax.lax.dot_general(
        jnp.ones((1, DIM), jnp.float32), wp * wp, dims,
        preferred_element_type=jnp.float32,
        precision=jax.lax.Precision.HIGHEST)

    d2 = (cross + (na + nw)
          + t * (2.0 * ab) + s * (2.0 * ac)
          + (t * t) * nb + (s * s) * nc + (2.0 * t * s) * bc)
    dist = jnp.sqrt(jnp.maximum(d2, 0.0))
    fc = (grow_ref[...] * _INV_SQRT2 + gcol_ref[...] * _INV_SQRT2
          - dist * _INV_SQRT2)

    e = ev_ref[...]
    b0 = b_ref[0:1, 0:1] * _INV_SQRT2
    b1 = b_ref[0:1, 1:2] * _INV_SQRT2
    b2 = b_ref[0:1, 2:3] * _INV_SQRT2
    th_hi = jnp.where(e == 1, b0, jnp.where(e == 2, b1, b2))
    th_lo = jnp.where(e == 2, b0, b1)
    erf_hi = jax.lax.erf(th_hi - fc)
    erf_lo = jnp.where(e == 1, -1.0, jax.lax.erf(th_lo - fc))
    p = 0.5 * (erf_hi - erf_lo)
    ll = jnp.where(e == 0, 0.0, jnp.log(p))
    partial = -jnp.sum(ll, axis=(0, 1), keepdims=True)

    @pl.when(pl.program_id(0) == 0)
    def _init():
        out_ref[...] = partial

    @pl.when(pl.program_id(0) != 0)
    def _acc():
        out_ref[...] += partial


@functools.partial(jax.jit, static_argnames=())
def kernel(events, times, z_rows, z_cols, gamma_rows, gamma_cols, b):
    z48 = jnp.transpose(z_rows, (1, 0, 2)).reshape(ROW_SIZE, 3 * DIM)
    out = pl.pallas_call(
        _nll_kernel,
        grid=(ROW_SIZE // BLK,),
        in_specs=[
            pl.BlockSpec((BLK, COL_SIZE), lambda i: (i, 0)),
            pl.BlockSpec((1, COL_SIZE), lambda i: (0, 0)),
            pl.BlockSpec((BLK, 3 * DIM), lambda i: (i, 0)),
            pl.BlockSpec((DIM, COL_SIZE), lambda i: (0, 0)),
            pl.BlockSpec((BLK, 1), lambda i: (i, 0)),
            pl.BlockSpec((1, COL_SIZE), lambda i: (0, 0)),
            pl.BlockSpec((1, 3), lambda i: (0, 0)),
        ],
        out_specs=pl.BlockSpec((1, 1), lambda i: (0, 0)),
        out_shape=jax.ShapeDtypeStruct((1, 1), jnp.float32),
    )(events, times.reshape(1, COL_SIZE), z48, z_cols.T,
      gamma_rows.reshape(ROW_SIZE, 1), gamma_cols.reshape(1, COL_SIZE),
      b.reshape(1, 3))
    return out[0, 0]
